# baseline (speedup 1.0000x reference)
import functools

import jax
import jax.numpy as jnp
from jax import lax
from jax.experimental import pallas as pl
from jax.experimental.pallas import tpu as pltpu
from jax.experimental.pallas import tpu_sc as plsc

B = 16384
EMB = 64
CHUNK = 16
NBUF = 2


def _sc_gather_rows(ut, it, us, isv, u3, i3):
    info = plsc.get_sparse_core_info()
    nc, ns = info.num_cores, info.num_subcores
    nw = nc * ns
    bpw = B // nw
    nch = bpw // CHUNK
    assert nch % NBUF == 0

    mesh = plsc.VectorSubcoreMesh(core_axis_name="c", subcore_axis_name="s")

    @functools.partial(
        pl.kernel,
        mesh=mesh,
        out_type=[
            jax.ShapeDtypeStruct((B, 128), jnp.float32),
            jax.ShapeDtypeStruct((B, 128), jnp.float32),
        ],
        scratch_types=[
            pltpu.VMEM((bpw,), jnp.int32),
            pltpu.VMEM((bpw,), jnp.int32),
            pltpu.VMEM((bpw,), jnp.int32),
            pltpu.VMEM((bpw,), jnp.int32),
            pltpu.VMEM((NBUF, CHUNK, 8, EMB), jnp.float32),
            pltpu.VMEM((NBUF, CHUNK, 8, EMB), jnp.float32),
            pltpu.VMEM((NBUF, CHUNK, 128), jnp.float32),
            pltpu.VMEM((NBUF, CHUNK, 128), jnp.float32),
            [pltpu.SemaphoreType.DMA] * NBUF,
            [pltpu.SemaphoreType.DMA] * NBUF,
            [pltpu.SemaphoreType.DMA] * NBUF,
            [pltpu.SemaphoreType.DMA] * NBUF,
        ],
    )
    def gather_kernel(ut_h, it_h, us_h, is_h, u3_h, i3_h, uo_h, io_h,
                      uix, iix, usx, isx, utl, itl, uob, iob,
                      sgu, sgi, swu, swi):
        wid = lax.axis_index("s") * nc + lax.axis_index("c")
        base = wid * bpw
        pltpu.sync_copy(ut_h.at[pl.ds(base, bpw)], uix)
        pltpu.sync_copy(it_h.at[pl.ds(base, bpw)], iix)
        pltpu.sync_copy(us_h.at[pl.ds(base, bpw)], usx)
        pltpu.sync_copy(is_h.at[pl.ds(base, bpw)], isx)

        u3 = u3_h.reshape(u3_h.shape[0] // 8, 8, EMB)
        i3 = i3_h.reshape(i3_h.shape[0] // 8, 8, EMB)
        tables = ((uix, usx, u3, utl, uob, sgu, swu, uo_h),
                  (iix, isx, i3, itl, iob, sgi, swi, io_h))

        def fire_gathers(j, b):
            for ix, sx, tab, tiles, obuf, sg, sw, out in tables:
                @pl.when(j >= NBUF)
                def _():
                    pltpu.make_async_copy(
                        obuf.at[b],
                        out.at[pl.ds(base + j * CHUNK, CHUNK)], sw[b]).wait()
                vec = ix[pl.ds(j * CHUNK, CHUNK)]
                for q in range(CHUNK):
                    pltpu.async_copy(tab.at[vec[q]], tiles.at[b, q], sg[b])

        def select_and_writeback(j, b):
            for ix, sx, tab, tiles, obuf, sg, sw, out in tables:
                pltpu.make_async_copy(tab.at[0], tiles.at[b], sg[b]).wait()
                sub = sx[pl.ds(j * CHUNK, CHUNK)]
                for q in range(CHUNK):
                    r = sub[q]
                    for c in range(EMB // 16):
                        obuf[b, q, pl.ds(c * 16, 16)] = (
                            tiles[b, q, r, pl.ds(c * 16, 16)])
                pltpu.async_copy(
                    obuf.at[b], out.at[pl.ds(base + j * CHUNK, CHUNK)], sw[b])

        def loop_body(jj):
            for b in range(NBUF):
                j = jj * NBUF + b
                fire_gathers(j, b)
                bp = (b - 1) % NBUF
                @pl.when(j >= 1)
                def _():
                    select_and_writeback(j - 1, bp)

        pl.loop(0, nch // NBUF)(loop_body)
        select_and_writeback(nch - 1, (nch - 1) % NBUF)
        for b in range(NBUF):
            for ix, sx, tab, tiles, obuf, sg, sw, out in tables:
                pltpu.make_async_copy(
                    obuf.at[b], out.at[pl.ds(base, CHUNK)], sw[b]).wait()

    return gather_kernel(ut, it, us, isv, u3, i3)


def _tc_mlp(ur, ir, gid, did, yid,
            gemb, demb, yemb, w1u, w1i, w1g, w1d, w1y, b1, w2, b2, w3t, b3):
    bsize = 1024
    nb = B // bsize

    def body(ur_, ir_, gi_, di_, yi_, ge_, de_, ye_,
             w1u_, w1i_, w1g_, w1d_, w1y_, b1_, w2_, b2_, w3_, b3_, o_):
        u = ur_[:, :EMB]
        iv = ir_[:, :EMB]
        ohg = (gi_[...] == lax.broadcasted_iota(jnp.int32, (bsize, 16), 1))
        ohd = (di_[...] == lax.broadcasted_iota(jnp.int32, (bsize, 32), 1))
        ohy = (yi_[...] == lax.broadcasted_iota(jnp.int32, (bsize, 64), 1))
        g8 = jnp.dot(ohg.astype(jnp.float32), ge_[...],
                     preferred_element_type=jnp.float32)
        d8 = jnp.dot(ohd.astype(jnp.float32), de_[...],
                     preferred_element_type=jnp.float32)
        y8 = jnp.dot(ohy.astype(jnp.float32), ye_[...],
                     preferred_element_type=jnp.float32)
        h = (jnp.dot(u, w1u_[...], preferred_element_type=jnp.float32)
             + jnp.dot(iv, w1i_[...], preferred_element_type=jnp.float32)
             + jnp.dot(g8, w1g_[...], preferred_element_type=jnp.float32)
             + jnp.dot(d8, w1d_[...], preferred_element_type=jnp.float32)
             + jnp.dot(y8, w1y_[...], preferred_element_type=jnp.float32)
             + b1_[...])
        h = jnp.maximum(h, 0.0)
        h2 = jnp.maximum(
            jnp.dot(h, w2_[...], preferred_element_type=jnp.float32) + b2_[...], 0.0)
        z = jnp.sum(h2 * w3_[...], axis=1, keepdims=True) + b3_[...]
        o_[...] = 1.0 / (1.0 + jnp.exp(-z))

    row = lambda i: (i, 0)
    rep = lambda i: (0, 0)
    return pl.pallas_call(
        body,
        grid=(nb,),
        in_specs=[
            pl.BlockSpec((bsize, 128), row),
            pl.BlockSpec((bsize, 128), row),
            pl.BlockSpec((bsize, 1), row),
            pl.BlockSpec((bsize, 1), row),
            pl.BlockSpec((bsize, 1), row),
            pl.BlockSpec((16, 8), rep),
            pl.BlockSpec((32, 8), rep),
            pl.BlockSpec((64, 8), rep),
            pl.BlockSpec((EMB, 128), rep),
            pl.BlockSpec((EMB, 128), rep),
            pl.BlockSpec((8, 128), rep),
            pl.BlockSpec((8, 128), rep),
            pl.BlockSpec((8, 128), rep),
            pl.BlockSpec((1, 128), rep),
            pl.BlockSpec((128, 64), rep),
            pl.BlockSpec((1, 64), rep),
            pl.BlockSpec((1, 64), rep),
            pl.BlockSpec((1, 1), rep),
        ],
        out_specs=pl.BlockSpec((bsize, 1), row),
        out_shape=jax.ShapeDtypeStruct((B, 1), jnp.float32),
    )(ur, ir, gid, did, yid, gemb, demb, yemb,
      w1u, w1i, w1g, w1d, w1y, b1, w2, b2, w3t, b3)


def kernel(user_ids, item_ids, genre_ids, director_ids, year_ids,
           user_emb, item_emb, genre_emb, director_emb, year_emb,
           W1, b1, W2, b2, W3, b3):
    uid = user_ids.astype(jnp.int32)
    iid = item_ids.astype(jnp.int32)
    ut = uid // 8
    it = iid // 8
    us = uid % 8
    isv = iid % 8

    ur, ir = _sc_gather_rows(ut, it, us, isv, user_emb, item_emb)

    gid = genre_ids.astype(jnp.int32).reshape(B, 1)
    did = director_ids.astype(jnp.int32).reshape(B, 1)
    yid = year_ids.astype(jnp.int32).reshape(B, 1)

    gemb = jnp.pad(genre_emb, ((0, 1), (0, 0)))
    demb = jnp.pad(director_emb, ((0, 2), (0, 0)))
    yemb = jnp.pad(year_emb, ((0, 14), (0, 0)))

    w1u = W1[0:EMB]
    w1i = W1[EMB:2 * EMB]
    w1g = W1[128:136]
    w1d = W1[136:144]
    w1y = W1[144:152]

    out = _tc_mlp(ur, ir, gid, did, yid, gemb, demb, yemb,
                  w1u, w1i, w1g, w1d, w1y,
                  b1.reshape(1, 128), W2, b2.reshape(1, 64),
                  W3.reshape(1, 64), b3.reshape(1, 1))
    return out.reshape(B)


# speedup vs baseline: 1.3834x; 1.3834x over previous
import functools

import jax
import jax.numpy as jnp
from jax import lax
from jax.experimental import pallas as pl
from jax.experimental.pallas import tpu as pltpu
from jax.experimental.pallas import tpu_sc as plsc

B = 16384
EMB = 64
CHUNK = 16
NBUF = 2


def _sc_gather_rows(uid, iid, u3, i3):
    info = plsc.get_sparse_core_info()
    nc, ns = info.num_cores, info.num_subcores
    nw = nc * ns
    bpw = B // nw
    nch = bpw // CHUNK
    assert nch % NBUF == 0

    mesh = plsc.VectorSubcoreMesh(core_axis_name="c", subcore_axis_name="s")

    @functools.partial(
        pl.kernel,
        mesh=mesh,
        out_type=jax.ShapeDtypeStruct((B, 128), jnp.float32),
        scratch_types=[
            pltpu.VMEM((bpw,), jnp.int32),
            pltpu.VMEM((bpw,), jnp.int32),
            pltpu.VMEM((NBUF, CHUNK, 8, EMB), jnp.float32),
            pltpu.VMEM((NBUF, CHUNK, 8, EMB), jnp.float32),
            pltpu.VMEM((NBUF, CHUNK, 128), jnp.float32),
            [pltpu.SemaphoreType.DMA] * NBUF,
            [pltpu.SemaphoreType.DMA] * NBUF,
            [pltpu.SemaphoreType.DMA] * NBUF,
        ],
    )
    def gather_kernel(uid_h, iid_h, u3_h, i3_h, o_h,
                      uix, iix, utl, itl, obuf, sgu, sgi, sw):
        wid = lax.axis_index("s") * nc + lax.axis_index("c")
        base = wid * bpw
        pltpu.sync_copy(uid_h.at[pl.ds(base, bpw)], uix)
        pltpu.sync_copy(iid_h.at[pl.ds(base, bpw)], iix)

        tables = ((uix, u3_h, utl, sgu, 0),
                  (iix, i3_h, itl, sgi, EMB))

        def fire_gathers(j, b):
            for ix, tab, tiles, sg, _ in tables:
                vec = ix[pl.ds(j * CHUNK, CHUNK)] >> 3
                for q in range(CHUNK):
                    pltpu.async_copy(tab.at[vec[q]], tiles.at[b, q], sg[b])

        def select_and_writeback(j, b):
            @pl.when(j >= NBUF)
            def _():
                pltpu.make_async_copy(
                    obuf.at[b],
                    o_h.at[pl.ds(base + j * CHUNK, CHUNK)], sw[b]).wait()
            for ix, tab, tiles, sg, col in tables:
                pltpu.make_async_copy(tab.at[0], tiles.at[b], sg[b]).wait()
                sub = ix[pl.ds(j * CHUNK, CHUNK)] & 7
                for q in range(CHUNK):
                    r = sub[q]
                    for c in range(EMB // 16):
                        obuf[b, q, pl.ds(col + c * 16, 16)] = (
                            tiles[b, q, r, pl.ds(c * 16, 16)])
            pltpu.async_copy(
                obuf.at[b], o_h.at[pl.ds(base + j * CHUNK, CHUNK)], sw[b])

        def loop_body(jj):
            for b in range(NBUF):
                j = jj * NBUF + b
                fire_gathers(j, b)
                bp = (b - 1) % NBUF
                @pl.when(j >= 1)
                def _():
                    select_and_writeback(j - 1, bp)

        pl.loop(0, nch // NBUF)(loop_body)
        select_and_writeback(nch - 1, (nch - 1) % NBUF)
        for b in range(NBUF):
            pltpu.make_async_copy(
                obuf.at[b], o_h.at[pl.ds(base, CHUNK)], sw[b]).wait()

    return gather_kernel(uid, iid, u3, i3)


def _tc_mlp(rows, gid, did, yid, gemb, demb, yemb, w1, b1, w2, b2, w3t, b3):
    bsize = 1024
    nb = B // bsize

    def body(rw_, gi_, di_, yi_, ge_, de_, ye_,
             w1_, b1_, w2_, b2_, w3_, b3_, o_):
        u = rw_[:, :EMB]
        iv = rw_[:, EMB:]
        ohg = (gi_[...] == lax.broadcasted_iota(jnp.int32, (bsize, 16), 1))
        ohd = (di_[...] == lax.broadcasted_iota(jnp.int32, (bsize, 32), 1))
        ohy = (yi_[...] == lax.broadcasted_iota(jnp.int32, (bsize, 64), 1))
        g8 = jnp.dot(ohg.astype(jnp.float32), ge_[...],
                     preferred_element_type=jnp.float32)
        d8 = jnp.dot(ohd.astype(jnp.float32), de_[...],
                     preferred_element_type=jnp.float32)
        y8 = jnp.dot(ohy.astype(jnp.float32), ye_[...],
                     preferred_element_type=jnp.float32)
        h = (jnp.dot(u, w1_[0:64, :], preferred_element_type=jnp.float32)
             + jnp.dot(iv, w1_[64:128, :], preferred_element_type=jnp.float32)
             + jnp.dot(g8, w1_[128:136, :], preferred_element_type=jnp.float32)
             + jnp.dot(d8, w1_[136:144, :], preferred_element_type=jnp.float32)
             + jnp.dot(y8, w1_[144:152, :], preferred_element_type=jnp.float32)
             + b1_[...])
        h = jnp.maximum(h, 0.0)
        h2 = jnp.maximum(
            jnp.dot(h, w2_[...], preferred_element_type=jnp.float32) + b2_[...], 0.0)
        z = jnp.sum(h2 * w3_[...], axis=1, keepdims=True) + b3_[...]
        o_[...] = 1.0 / (1.0 + jnp.exp(-z))

    row = lambda i: (i, 0)
    rep = lambda i: (0, 0)
    return pl.pallas_call(
        body,
        grid=(nb,),
        in_specs=[
            pl.BlockSpec((bsize, 128), row),
            pl.BlockSpec((bsize, 1), row),
            pl.BlockSpec((bsize, 1), row),
            pl.BlockSpec((bsize, 1), row),
            pl.BlockSpec((16, 8), rep),
            pl.BlockSpec((32, 8), rep),
            pl.BlockSpec((64, 8), rep),
            pl.BlockSpec((152, 128), rep),
            pl.BlockSpec((1, 128), rep),
            pl.BlockSpec((128, 64), rep),
            pl.BlockSpec((1, 64), rep),
            pl.BlockSpec((1, 64), rep),
            pl.BlockSpec((1, 1), rep),
        ],
        out_specs=pl.BlockSpec((bsize, 1), row),
        out_shape=jax.ShapeDtypeStruct((B, 1), jnp.float32),
    )(rows, gid, did, yid, gemb, demb, yemb, w1, b1, w2, b2, w3t, b3)


def kernel(user_ids, item_ids, genre_ids, director_ids, year_ids,
           user_emb, item_emb, genre_emb, director_emb, year_emb,
           W1, b1, W2, b2, W3, b3):
    uid = user_ids.astype(jnp.int32)
    iid = item_ids.astype(jnp.int32)
    u3 = user_emb.reshape(user_emb.shape[0] // 8, 8, EMB)
    i3 = item_emb.reshape(item_emb.shape[0] // 8, 8, EMB)

    rows = _sc_gather_rows(uid, iid, u3, i3)

    gid = genre_ids.astype(jnp.int32).reshape(B, 1)
    did = director_ids.astype(jnp.int32).reshape(B, 1)
    yid = year_ids.astype(jnp.int32).reshape(B, 1)

    gemb = jnp.pad(genre_emb, ((0, 1), (0, 0)))
    demb = jnp.pad(director_emb, ((0, 2), (0, 0)))
    yemb = jnp.pad(year_emb, ((0, 14), (0, 0)))

    out = _tc_mlp(rows, gid, did, yid, gemb, demb, yemb, W1,
                  b1.reshape(1, 128), W2, b2.reshape(1, 64),
                  W3.reshape(1, 64), b3.reshape(1, 1))
    return out.reshape(B)


# speedup vs baseline: 1.4251x; 1.0302x over previous
import functools

import jax
import jax.numpy as jnp
from jax import lax
from jax.experimental import pallas as pl
from jax.experimental.pallas import tpu as pltpu
from jax.experimental.pallas import tpu_sc as plsc

B = 16384
EMB = 64
CHUNK = 16
NBUF = 2


def _sc_gather_rows(uid, iid, u3, i3):
    info = plsc.get_sparse_core_info()
    nc, ns = info.num_cores, info.num_subcores
    nw = nc * ns
    bpw = B // nw
    nch = bpw // CHUNK
    assert nch % NBUF == 0

    mesh = plsc.VectorSubcoreMesh(core_axis_name="c", subcore_axis_name="s")

    @functools.partial(
        pl.kernel,
        mesh=mesh,
        out_type=jax.ShapeDtypeStruct((B, 128), jnp.float32),
        scratch_types=[
            pltpu.VMEM((bpw,), jnp.int32),
            pltpu.VMEM((bpw,), jnp.int32),
            pltpu.VMEM((NBUF, CHUNK, 8, EMB), jnp.float32),
            pltpu.VMEM((NBUF, CHUNK, 8, EMB), jnp.float32),
            pltpu.VMEM((NBUF, CHUNK, 128), jnp.float32),
            [pltpu.SemaphoreType.DMA] * NBUF,
            [pltpu.SemaphoreType.DMA] * NBUF,
            [pltpu.SemaphoreType.DMA] * NBUF,
        ],
    )
    def gather_kernel(uid_h, iid_h, u3_h, i3_h, o_h,
                      uix, iix, utl, itl, obuf, sgu, sgi, sw):
        wid = lax.axis_index("s") * nc + lax.axis_index("c")
        base = wid * bpw
        pltpu.sync_copy(uid_h.at[pl.ds(base, bpw)], uix)
        pltpu.sync_copy(iid_h.at[pl.ds(base, bpw)], iix)

        tables = ((uix, u3_h, utl, sgu, 0),
                  (iix, i3_h, itl, sgi, EMB))

        def fire_gathers(j, b):
            for ix, tab, tiles, sg, _ in tables:
                vec = ix[pl.ds(j * CHUNK, CHUNK)] >> 3
                for q in range(CHUNK):
                    pltpu.async_copy(tab.at[vec[q]], tiles.at[b, q], sg[b])

        def select_and_writeback(j, b):
            @pl.when(j >= NBUF)
            def _():
                pltpu.make_async_copy(
                    obuf.at[b],
                    o_h.at[pl.ds(base + j * CHUNK, CHUNK)], sw[b]).wait()
            for ix, tab, tiles, sg, col in tables:
                pltpu.make_async_copy(tab.at[0], tiles.at[b], sg[b]).wait()
                sub = ix[pl.ds(j * CHUNK, CHUNK)] & 7
                for q in range(CHUNK):
                    r = sub[q]
                    for c in range(EMB // 16):
                        obuf[b, q, pl.ds(col + c * 16, 16)] = (
                            tiles[b, q, r, pl.ds(c * 16, 16)])
            pltpu.async_copy(
                obuf.at[b], o_h.at[pl.ds(base + j * CHUNK, CHUNK)], sw[b])

        def loop_body(jj):
            for b in range(NBUF):
                j = jj * NBUF + b
                fire_gathers(j, b)
                bp = (b - 1) % NBUF
                @pl.when(j >= 1)
                def _():
                    select_and_writeback(j - 1, bp)

        pl.loop(0, nch // NBUF)(loop_body)
        select_and_writeback(nch - 1, (nch - 1) % NBUF)
        for b in range(NBUF):
            pltpu.make_async_copy(
                obuf.at[b], o_h.at[pl.ds(base, CHUNK)], sw[b]).wait()

    return gather_kernel(uid, iid, u3, i3)


def _tc_mlp(rows, ids, gemb, demb, yemb, w1, b1, w2, b2, w3t, b3):
    bsize = 1024
    nb = B // bsize

    tdot = lambda a, b: lax.dot_general(
        a, b, (((0,), (0,)), ((), ())), preferred_element_type=jnp.float32)

    def body(rw_, ids_, ge_, de_, ye_,
             w1_, b1_, w2_, b2_, w3_, b3_, o_):
        u = rw_[:, :EMB]
        iv = rw_[:, EMB:]
        ge = jnp.concatenate([ge_[...], jnp.zeros((1, 8), jnp.float32)], 0)
        de = jnp.concatenate([de_[...], jnp.zeros((2, 8), jnp.float32)], 0)
        ye = jnp.concatenate([ye_[...], jnp.zeros((14, 8), jnp.float32)], 0)
        gi = ids_[0:1, :]
        di = ids_[1:2, :]
        yi = ids_[2:3, :]
        ohg = (jnp.broadcast_to(gi, (16, bsize))
               == lax.broadcasted_iota(jnp.int32, (16, bsize), 0))
        ohd = (jnp.broadcast_to(di, (32, bsize))
               == lax.broadcasted_iota(jnp.int32, (32, bsize), 0))
        ohy = (jnp.broadcast_to(yi, (64, bsize))
               == lax.broadcasted_iota(jnp.int32, (64, bsize), 0))
        g8 = tdot(ohg.astype(jnp.float32), ge)
        d8 = tdot(ohd.astype(jnp.float32), de)
        y8 = tdot(ohy.astype(jnp.float32), ye)
        h = (jnp.dot(u, w1_[0:64, :], preferred_element_type=jnp.float32)
             + jnp.dot(iv, w1_[64:128, :], preferred_element_type=jnp.float32)
             + jnp.dot(g8, w1_[128:136, :], preferred_element_type=jnp.float32)
             + jnp.dot(d8, w1_[136:144, :], preferred_element_type=jnp.float32)
             + jnp.dot(y8, w1_[144:152, :], preferred_element_type=jnp.float32)
             + b1_[...])
        h = jnp.maximum(h, 0.0)
        h2 = jnp.maximum(
            jnp.dot(h, w2_[...], preferred_element_type=jnp.float32) + b2_[...], 0.0)
        z = jnp.sum(h2 * w3_[...], axis=1, keepdims=True) + b3_[...]
        o_[...] = 1.0 / (1.0 + jnp.exp(-z))

    row = lambda i: (i, 0)
    rep = lambda i: (0, 0)
    return pl.pallas_call(
        body,
        grid=(nb,),
        in_specs=[
            pl.BlockSpec((bsize, 128), row),
            pl.BlockSpec((8, bsize), lambda i: (0, i)),
            pl.BlockSpec((15, 8), rep),
            pl.BlockSpec((30, 8), rep),
            pl.BlockSpec((50, 8), rep),
            pl.BlockSpec((152, 128), rep),
            pl.BlockSpec((1, 128), rep),
            pl.BlockSpec((128, 64), rep),
            pl.BlockSpec((1, 64), rep),
            pl.BlockSpec((1, 64), rep),
            pl.BlockSpec((1, 1), rep),
        ],
        out_specs=pl.BlockSpec((bsize, 1), row),
        out_shape=jax.ShapeDtypeStruct((B, 1), jnp.float32),
    )(rows, ids, gemb, demb, yemb, w1, b1, w2, b2, w3t, b3)


def kernel(user_ids, item_ids, genre_ids, director_ids, year_ids,
           user_emb, item_emb, genre_emb, director_emb, year_emb,
           W1, b1, W2, b2, W3, b3):
    uid = user_ids.astype(jnp.int32)
    iid = item_ids.astype(jnp.int32)
    u3 = user_emb.reshape(user_emb.shape[0] // 8, 8, EMB)
    i3 = item_emb.reshape(item_emb.shape[0] // 8, 8, EMB)

    rows = _sc_gather_rows(uid, iid, u3, i3)

    ids = jnp.zeros((8, B), jnp.int32)
    ids = ids.at[0].set(genre_ids.astype(jnp.int32))
    ids = ids.at[1].set(director_ids.astype(jnp.int32))
    ids = ids.at[2].set(year_ids.astype(jnp.int32))

    out = _tc_mlp(rows, ids, genre_emb, director_emb, year_emb, W1,
                  b1.reshape(1, 128), W2, b2.reshape(1, 64),
                  W3.reshape(1, 64), b3.reshape(1, 1))
    return out.reshape(B)
